# baseline (speedup 1.0000x reference)
import functools

import jax
import jax.numpy as jnp
from jax import lax
from jax.experimental import pallas as pl
from jax.experimental.pallas import tpu as pltpu
from jax.experimental.pallas import tpu_sc as plsc

_NC = 2
_NS = 16
_NW = _NC * _NS
_NBUF = 4


def kernel(weight, input):
    V, D = weight.shape
    (B,) = input.shape
    wT = weight.T
    b_per_w = B // _NW
    mesh = plsc.VectorSubcoreMesh(core_axis_name="c", subcore_axis_name="s")

    @functools.partial(
        pl.kernel,
        mesh=mesh,
        out_type=jax.ShapeDtypeStruct((D, B), jnp.float32),
        scratch_types=[
            pltpu.VMEM((b_per_w,), jnp.int32),
            pltpu.VMEM((D, 128), jnp.float32),
            pltpu.VMEM((D, 128), jnp.float32),
            pltpu.VMEM((D, 128), jnp.float32),
            pltpu.VMEM((D, 128), jnp.float32),
            pltpu.VMEM((4, 8, b_per_w), jnp.float32),
            pltpu.SemaphoreType.DMA,
            pltpu.SemaphoreType.DMA,
            pltpu.SemaphoreType.DMA,
            pltpu.SemaphoreType.DMA,
        ],
        compiler_params=pltpu.CompilerParams(needs_layout_passes=False),
    )
    def _gather(
        tableT_hbm, idx_hbm, outT_hbm,
        idx_v, b0, b1, b2, b3, acc, s0, s1, s2, s3,
    ):
        bufs = [b0, b1, b2, b3]
        sems = [s0, s1, s2, s3]
        wid = lax.axis_index("s") * _NC + lax.axis_index("c")
        base = wid * b_per_w
        pltpu.sync_copy(idx_hbm.at[pl.ds(base, b_per_w)], idx_v)

        lane = lax.iota(jnp.int32, 16)
        s_lo = lane // 8
        dd = lane % 8

        def fire(r, j):
            off = pl.multiple_of((r >> 7) << 7, 128)
            pltpu.async_copy(
                tableT_hbm.at[:, pl.ds(off, 128)], bufs[j], sems[j]
            )

        def drain(j):
            pltpu.make_async_copy(
                tableT_hbm.at[:, pl.ds(0, 128)], bufs[j], sems[j]
            ).wait()

        def gather(rr16, p16, j):
            for h in range(2):
                col = plsc.load_gather(bufs[j], [lane + 16 * h, rr16])
                plsc.store_scatter(acc, [s_lo + 2 * h, dd, p16], col)

        def body(g, carry):
            i0 = g * 16
            vec = idx_v[pl.ds(i0, 16)]
            rs = [
                jnp.max(jnp.where(lane == k, vec, 0))
                for k in range(16)
            ]
            for k in range(_NBUF - 1):
                fire(rs[k], k)
            for k in range(16):
                if k + _NBUF - 1 < 16:
                    fire(rs[k + _NBUF - 1], (k + _NBUF - 1) % _NBUF)
                drain(k % _NBUF)
                rr16 = jnp.full((16,), rs[k] & 127, jnp.int32)
                p16 = jnp.full((16,), i0 + k, jnp.int32)
                gather(rr16, p16, k % _NBUF)
            return carry

        lax.fori_loop(0, b_per_w // 16, body, 0)
        for s in range(4):
            pltpu.sync_copy(
                acc.at[s],
                outT_hbm.at[pl.ds(8 * s, 8), pl.ds(base, b_per_w)],
            )

    outT = _gather(wT, input)
    return outT.T


# speedup vs baseline: 1.3027x; 1.3027x over previous
import functools

import jax
import jax.numpy as jnp
from jax import lax
from jax.experimental import pallas as pl
from jax.experimental.pallas import tpu as pltpu
from jax.experimental.pallas import tpu_sc as plsc

_NC = 2
_NS = 16
_NW = _NC * _NS
_NBUF = 8
_G = 16


def kernel(weight, input):
    V, D = weight.shape
    (B,) = input.shape
    wT = weight.T
    b_per_w = B // _NW
    n_groups = b_per_w // _G
    mesh = plsc.VectorSubcoreMesh(core_axis_name="c", subcore_axis_name="s")

    @functools.partial(
        pl.kernel,
        mesh=mesh,
        out_type=jax.ShapeDtypeStruct((D, B), jnp.float32),
        scratch_types=[
            pltpu.VMEM((b_per_w + _G,), jnp.int32),
        ]
        + [pltpu.VMEM((D, 128), jnp.float32) for _ in range(_NBUF)]
        + [
            pltpu.VMEM((4, 8, b_per_w), jnp.float32),
        ]
        + [pltpu.SemaphoreType.DMA for _ in range(_NBUF)],
        compiler_params=pltpu.CompilerParams(needs_layout_passes=False),
    )
    def _gather(tableT_hbm, idx_hbm, outT_hbm, idx_v, *rest):
        bufs = list(rest[:_NBUF])
        acc = rest[_NBUF]
        sems = list(rest[_NBUF + 1:])
        wid = lax.axis_index("s") * _NC + lax.axis_index("c")
        base = wid * b_per_w
        pltpu.sync_copy(idx_hbm.at[pl.ds(base, b_per_w)],
                        idx_v.at[pl.ds(0, b_per_w)])

        lane = lax.iota(jnp.int32, 16)
        s_lo = lane // 8
        dd = lane % 8

        def extract(vec, k):
            return jnp.max(jnp.where(lane == k, vec, 0))

        def fire(r, j):
            off = pl.multiple_of((r >> 7) << 7, 128)
            pltpu.async_copy(
                tableT_hbm.at[:, pl.ds(off, 128)], bufs[j], sems[j]
            )

        def drain(j):
            pltpu.make_async_copy(
                tableT_hbm.at[:, pl.ds(0, 128)], bufs[j], sems[j]
            ).wait()

        def gather(rr16, p16, j):
            for h in range(2):
                col = plsc.load_gather(bufs[j], [lane + 16 * h, rr16])
                plsc.store_scatter(acc, [s_lo + 2 * h, dd, p16], col)

        _AHEAD = _NBUF - 1
        vec0 = idx_v[pl.ds(0, _G)]
        for k in range(_AHEAD):
            fire(extract(vec0, k), k % _NBUF)

        def body(g, carry):
            i0 = g * _G
            vec = idx_v[pl.ds(i0, _G)]
            rs = [extract(vec, k) for k in range(_G)]
            vecn = idx_v[pl.ds(i0 + _G, _G)]
            live = (g + 1) < n_groups
            rn = [
                jnp.where(live, extract(vecn, k), 0)
                for k in range(_AHEAD)
            ]
            for k in range(_G):
                ka = k + _AHEAD
                r_ahead = rs[ka] if ka < _G else rn[ka - _G]
                fire(r_ahead, ka % _NBUF)
                drain(k % _NBUF)
                rr16 = jnp.full((16,), rs[k] & 127, jnp.int32)
                p16 = jnp.full((16,), i0 + k, jnp.int32)
                gather(rr16, p16, k % _NBUF)
            return carry

        lax.fori_loop(0, n_groups, body, 0)
        for j in range(_AHEAD):
            drain((b_per_w + j) % _NBUF)
        for s in range(4):
            pltpu.sync_copy(
                acc.at[s],
                outT_hbm.at[pl.ds(8 * s, 8), pl.ds(base, b_per_w)],
            )

    outT = _gather(wT, input)
    return outT.T
